# Initial kernel scaffold; baseline (speedup 1.0000x reference)
import jax
import jax.numpy as jnp
from jax.experimental import pallas as pl


def kernel(unknown, known, known_feats):
    raise NotImplementedError("write your pallas kernel here")


# speedup vs baseline: 271.3898x; 271.3898x over previous
import jax
import jax.numpy as jnp
from jax.experimental import pallas as pl

_B, _N, _M, _C = 4, 8192, 2048, 64
_TN = 512


def _knn_interp_body(u_ref, k_ref, f_ref, out_ref):
    u = u_ref[0]
    kp = k_ref[0]

    d2 = jnp.zeros((_TN, _M), jnp.float32)
    for d in range(3):
        diff = u[:, d][:, None] - kp[d, :][None, :]
        d2 = d2 + diff * diff

    iota = jax.lax.broadcasted_iota(jnp.int32, (_TN, _M), 1)
    dcur = d2
    vals, idxs = [], []
    for k in range(3):
        mn = jnp.min(dcur, axis=1, keepdims=True)
        am = jnp.min(jnp.where(dcur == mn, iota, _M), axis=1, keepdims=True)
        vals.append(mn)
        idxs.append(am)
        if k < 2:
            dcur = jnp.where(iota == am, jnp.float32(jnp.inf), dcur)

    recips = [1.0 / (jnp.sqrt(jnp.maximum(v, 0.0)) + 1e-8) for v in vals]
    norm = (recips[0] + recips[1]) + recips[2]

    sel = jnp.zeros((_TN, _M), jnp.float32)
    for k in range(3):
        w = recips[k] / norm
        sel = sel + jnp.where(iota == idxs[k], w, 0.0)

    out_ref[0] = jax.lax.dot_general(
        f_ref[0], sel, (((1,), (1,)), ((), ())),
        preferred_element_type=jnp.float32)


def kernel(unknown, known, known_feats):
    known_t = jnp.transpose(known, (0, 2, 1))
    return pl.pallas_call(
        _knn_interp_body,
        grid=(_B, _N // _TN),
        in_specs=[
            pl.BlockSpec((1, _TN, 3), lambda b, i: (b, i, 0)),
            pl.BlockSpec((1, 3, _M), lambda b, i: (b, 0, 0)),
            pl.BlockSpec((1, _C, _M), lambda b, i: (b, 0, 0)),
        ],
        out_specs=pl.BlockSpec((1, _C, _TN), lambda b, i: (b, 0, i)),
        out_shape=jax.ShapeDtypeStruct((_B, _C, _N), jnp.float32),
    )(unknown, known_t, known_feats)
